# baseline (speedup 1.0000x reference)
import functools

import jax
import jax.numpy as jnp
from jax import lax
from jax.experimental import pallas as pl
from jax.experimental.pallas import tpu as pltpu
from jax.experimental.pallas import tpu_sc as plsc

_ROWS_PER_BLOCK = 512
_R_TC = 3584


def _matvec_body(w_ref, emb_ref, v_ref, l1_ref):
    v_ref[...] = jnp.sum(emb_ref[...] * w_ref[...], axis=1)

    @pl.when(pl.program_id(0) == 0)
    def _():
        l1_ref[...] = jnp.sum(jnp.abs(w_ref[...]), keepdims=True)


def _matvec(emb, w, n_rows):
    V, D = emb.shape
    nb = n_rows // _ROWS_PER_BLOCK
    return pl.pallas_call(
        _matvec_body,
        grid=(nb,),
        in_specs=[
            pl.BlockSpec((1, D), lambda i: (0, 0)),
            pl.BlockSpec((_ROWS_PER_BLOCK, D), lambda i: (i, 0)),
        ],
        out_specs=[
            pl.BlockSpec((_ROWS_PER_BLOCK,), lambda i: (i,)),
            pl.BlockSpec((1, 1), lambda i: (0, 0)),
        ],
        out_shape=[
            jax.ShapeDtypeStruct((n_rows,), jnp.float32),
            jax.ShapeDtypeStruct((1, 1), jnp.float32),
        ],
        compiler_params=pltpu.CompilerParams(
            dimension_semantics=("arbitrary",)
        ),
    )(w, emb)


@functools.lru_cache(maxsize=None)
def _make_sc_matvec(row0, n_rows, D):
    info = plsc.get_sparse_core_info()
    NC, NS, L = info.num_cores, info.num_subcores, info.num_lanes
    NW = NC * NS
    rpw = n_rows // NW
    nchunk = D // L
    mesh = plsc.VectorSubcoreMesh(core_axis_name="c", subcore_axis_name="s")

    @functools.partial(
        pl.kernel,
        mesh=mesh,
        out_type=jax.ShapeDtypeStruct((n_rows, L), jnp.float32),
        scratch_types=[
            pltpu.VMEM((D,), jnp.float32),
            pltpu.VMEM((rpw, D), jnp.float32),
            pltpu.VMEM((rpw, L), jnp.float32),
        ],
    )
    def sc_matvec(emb_hbm, w_hbm, p_hbm, w_v, rows_v, out_v):
        wid = lax.axis_index("s") * NC + lax.axis_index("c")
        my0 = wid * rpw
        pltpu.sync_copy(w_hbm, w_v)
        pltpu.sync_copy(emb_hbm.at[pl.ds(row0 + my0, rpw)], rows_v)

        for r0 in range(0, rpw, L):
            def chunk_body(c, accs):
                base = c * L
                wc = w_v[pl.ds(base, L)]
                return tuple(
                    accs[r] + rows_v[r0 + r, pl.ds(base, L)] * wc
                    for r in range(L)
                )

            accs = lax.fori_loop(
                0, nchunk, chunk_body,
                tuple(jnp.zeros((L,), jnp.float32) for _ in range(L)),
            )
            for r in range(L):
                out_v[r0 + r, :] = accs[r]

        pltpu.sync_copy(out_v, p_hbm.at[pl.ds(my0, rpw)])

    return sc_matvec


_GB = 512


def _tc_gather_body(va_ref, pb_ref, x_ref, out_ref, v2_ref):
    @pl.when(pl.program_id(0) == 0)
    def _():
        vb = jnp.sum(pb_ref[...], axis=1)
        v = jnp.concatenate([va_ref[...], vb])
        v2_ref[...] = v.reshape(32, 128)

    v2 = v2_ref[...]
    xb = x_ref[0]
    hi = xb >> 7
    lo = xb & 127
    oh = (
        lax.broadcasted_iota(jnp.int32, (32, _GB), 0) == hi
    ).astype(jnp.float32)
    t = lax.dot_general(
        v2, oh, (((0,), (0,)), ((), ())),
        preferred_element_type=jnp.float32,
    )
    lomask = lax.broadcasted_iota(jnp.int32, (128, _GB), 0) == lo
    g = jnp.sum(jnp.where(lomask, t, 0.0), axis=0, keepdims=True)
    out_ref[0] = g


def _tc_gather(va, pb, x3, B):
    nb = B // _GB
    na, nbv = va.shape[0], pb.shape[0]
    return pl.pallas_call(
        _tc_gather_body,
        grid=(nb,),
        in_specs=[
            pl.BlockSpec((na,), lambda j: (0,)),
            pl.BlockSpec((nbv, pb.shape[1]), lambda j: (0, 0)),
            pl.BlockSpec((1, 1, _GB), lambda j: (j, 0, 0)),
        ],
        out_specs=pl.BlockSpec((1, 1, _GB), lambda j: (j, 0, 0)),
        out_shape=jax.ShapeDtypeStruct((nb, 1, _GB), jnp.float32),
        scratch_shapes=[pltpu.VMEM((32, 128), jnp.float32)],
        compiler_params=pltpu.CompilerParams(
            dimension_semantics=("arbitrary",)
        ),
    )(va, pb, x3)


def kernel(x, emb, W):
    B = x.shape[0]
    V, D = emb.shape
    va, l1 = _matvec(emb, W, _R_TC)
    pb = _make_sc_matvec(_R_TC, V - _R_TC, D)(emb, W.reshape(D))
    x3 = x.astype(jnp.int32).reshape(B // _GB, 1, _GB)
    out = _tc_gather(va, pb, x3, B)
    return out.reshape(B, 1), l1[0, 0]


# speedup vs baseline: 1.3403x; 1.3403x over previous
import functools

import jax
import jax.numpy as jnp
from jax import lax
from jax.experimental import pallas as pl
from jax.experimental.pallas import tpu as pltpu
from jax.experimental.pallas import tpu_sc as plsc

_ROWS_PER_BLOCK = 512


def _matvec_body(w_ref, emb_ref, v_ref, l1_ref):
    v_ref[...] = jnp.sum(emb_ref[...] * w_ref[...], axis=1)

    @pl.when(pl.program_id(0) == 0)
    def _():
        l1_ref[...] = jnp.sum(jnp.abs(w_ref[...]), keepdims=True)


def _matvec(emb, w):
    V, D = emb.shape
    nb = V // _ROWS_PER_BLOCK
    return pl.pallas_call(
        _matvec_body,
        grid=(nb,),
        in_specs=[
            pl.BlockSpec((1, D), lambda i: (0, 0)),
            pl.BlockSpec((_ROWS_PER_BLOCK, D), lambda i: (i, 0)),
        ],
        out_specs=[
            pl.BlockSpec((_ROWS_PER_BLOCK,), lambda i: (i,)),
            pl.BlockSpec((1, 1), lambda i: (0, 0)),
        ],
        out_shape=[
            jax.ShapeDtypeStruct((V,), jnp.float32),
            jax.ShapeDtypeStruct((1, 1), jnp.float32),
        ],
        compiler_params=pltpu.CompilerParams(
            dimension_semantics=("arbitrary",)
        ),
    )(w, emb)


_IDX_ROW = 128


@functools.lru_cache(maxsize=None)
def _make_gather(B, V):
    info = plsc.get_sparse_core_info()
    NC, NS = info.num_cores, info.num_subcores
    NW = NC * NS
    bpw = B // NW
    kj = bpw // _IDX_ROW
    mesh = plsc.VectorSubcoreMesh(core_axis_name="c", subcore_axis_name="s")

    @functools.partial(
        pl.kernel,
        mesh=mesh,
        out_type=jax.ShapeDtypeStruct((NW, kj, _IDX_ROW), jnp.float32),
        scratch_types=[
            pltpu.VMEM((kj, _IDX_ROW), jnp.int32),
            pltpu.VMEM((kj, _IDX_ROW), jnp.float32),
            pltpu.SemaphoreType.DMA,
        ],
    )
    def gather_k(v_hbm, x_hbm, out_hbm, idx_v, out_v, sem):
        wid = lax.axis_index("s") * NC + lax.axis_index("c")
        pltpu.sync_copy(x_hbm.at[wid], idx_v)
        copies = [
            pltpu.async_copy(v_hbm.at[idx_v.at[j]], out_v.at[j], sem)
            for j in range(kj)
        ]
        for c in copies:
            c.wait()
        pltpu.sync_copy(out_v, out_hbm.at[wid])

    return gather_k, NW, kj


def kernel(x, emb, W):
    B = x.shape[0]
    V, D = emb.shape
    v, l1 = _matvec(emb, W)
    gather_k, NW, kj = _make_gather(B, V)
    x3 = x.astype(jnp.int32).reshape(NW, kj, _IDX_ROW)
    out = gather_k(v, x3)
    return out.reshape(B, 1), l1[0, 0]
